# baseline (speedup 1.0000x reference)
import functools
import math

import jax
import jax.numpy as jnp
import numpy as np
from jax import lax
from jax.experimental import pallas as pl
from jax.experimental.pallas import tpu as pltpu
from jax.experimental.pallas import tpu_sc as plsc

HS = 128
HBS = 32
RBF_DIM = 32
MAX_RADIUS = 15.0
NUM_LAYERS = 5
NUM_TYPES = 20
N = 10000
E = 320000

NC = 2
NS = 16
NW = NC * NS
EPW = E // NW
CH = 80
GW = HS // 2
NCH = EPW // CH
RPT = N // NS

_mesh = plsc.VectorSubcoreMesh(core_axis_name="c", subcore_axis_name="s")


@functools.partial(
    pl.kernel,
    out_type=jax.ShapeDtypeStruct((E,), jnp.float32),
    mesh=_mesh,
    scratch_types=[
        pltpu.VMEM((3 * N,), jnp.float32),
        pltpu.VMEM((EPW,), jnp.int32),
        pltpu.VMEM((EPW,), jnp.int32),
        pltpu.VMEM((EPW,), jnp.float32),
    ],
    compiler_params=pltpu.CompilerParams(needs_layout_passes=False),
)
def _geom(pos_hbm, dst_hbm, src_hbm, d2_hbm, pos_v, dst_v, src_v, d2_v):
    cid = lax.axis_index("c")
    sid = lax.axis_index("s")
    base = pl.multiple_of((cid * NS + sid) * EPW, 8)
    pltpu.sync_copy(pos_hbm, pos_v)
    pltpu.sync_copy(dst_hbm.at[pl.ds(base, EPW)], dst_v)
    pltpu.sync_copy(src_hbm.at[pl.ds(base, EPW)], src_v)

    @pl.loop(0, EPW // 16)
    def _(i):
        s = pl.ds(i * 16, 16)
        vd = dst_v[s]
        vs = src_v[s]
        dx = plsc.load_gather(pos_v, [vd]) - plsc.load_gather(pos_v, [vs])
        vd = vd + N
        vs = vs + N
        dy = plsc.load_gather(pos_v, [vd]) - plsc.load_gather(pos_v, [vs])
        vd = vd + N
        vs = vs + N
        dz = plsc.load_gather(pos_v, [vd]) - plsc.load_gather(pos_v, [vs])
        d2_v[s] = dx * dx + dy * dy + dz * dz

    pltpu.sync_copy(d2_v, d2_hbm.at[pl.ds(base, EPW)])


N_PAD = 10240
RPT_P = N_PAD // NS


@functools.partial(
    pl.kernel,
    out_type=jax.ShapeDtypeStruct((NC * N_PAD, HS), jnp.float32),
    mesh=_mesh,
    scratch_types=[
        pltpu.VMEM_SHARED((N_PAD, HS), jnp.float32),
        pltpu.VMEM((CH,), jnp.int32),
        pltpu.VMEM((CH,), jnp.int32),
        pltpu.VMEM((CH,), jnp.int32),
        pltpu.VMEM((CH,), jnp.int32),
        pltpu.VMEM((CH,), jnp.int32),
        pltpu.VMEM((CH,), jnp.int32),
        pltpu.VMEM((CH, HS), jnp.float32),
        pltpu.VMEM((CH, HS), jnp.float32),
        pltpu.VMEM((CH * GW,), jnp.int32),
        pltpu.VMEM((CH * GW,), jnp.int32),
        pltpu.SemaphoreType.DMA,
        pltpu.SemaphoreType.DMA,
        pltpu.SemaphoreType.DMA,
        pltpu.SemaphoreType.DMA,
        pltpu.SemaphoreType.DMA,
        pltpu.SemaphoreType.DMA,
    ],
    compiler_params=pltpu.CompilerParams(needs_layout_passes=False),
)
def _gms(hw_hbm, gate_hbm, dst_hbm, src_hbm, agg_hbm,
         agg_sh, dstb0, dstb1, srcb0, srcb1, dsts0, dsts1,
         rows0, rows1, gate0, gate1,
         semg0, semg1, semc0, semc1, sems0, sems1):
    cid = lax.axis_index("c")
    sid = lax.axis_index("s")
    wid = cid * NS + sid
    base_e = wid * EPW

    @pl.loop(0, CH)
    def _(r):
        for j in range(HS // 16):
            rows0[r, pl.ds(j * 16, 16)] = jnp.zeros((16,), jnp.float32)

    for i in range(RPT_P // CH):
        pltpu.sync_copy(
            rows0, agg_sh.at[pl.ds(pl.multiple_of(sid * RPT_P + i * CH, 8), CH)])
    plsc.subcore_barrier()

    def _issue_c(i, dstb, srcb, gate, semc):
        off = pl.multiple_of(base_e + i * CH, 8)
        pltpu.async_copy(dst_hbm.at[pl.ds(off, CH)], dstb, semc)
        pltpu.async_copy(src_hbm.at[pl.ds(off, CH)], srcb, semc)
        offg = pl.multiple_of((base_e + i * CH) * GW, 8)
        pltpu.async_copy(gate_hbm.at[pl.ds(offg, CH * GW)], gate, semc)

    def _wait_c(i, dstb, srcb, gate, semc):
        off = pl.multiple_of(base_e + i * CH, 8)
        pltpu.make_async_copy(dst_hbm.at[pl.ds(off, CH)], dstb, semc).wait()
        pltpu.make_async_copy(src_hbm.at[pl.ds(off, CH)], srcb, semc).wait()
        offg = pl.multiple_of((base_e + i * CH) * GW, 8)
        pltpu.make_async_copy(gate_hbm.at[pl.ds(offg, CH * GW)], gate, semc).wait()

    def _issue_g(srcb, rows, semg):
        pltpu.async_copy(hw_hbm.at[srcb], rows, semg)

    def _wait_g(srcb, rows, semg):
        pltpu.make_async_copy(hw_hbm.at[srcb], rows, semg).wait()

    def _proc(dstb, dsts, rows, gate, sems):
        for j in range(CH // 16):
            sl = pl.ds(j * 16, 16)
            dsts[sl] = dstb[sl]

        @pl.loop(0, CH)
        def _(r):
            for j in range(GW // 16):
                g = gate[pl.ds(r * GW + j * 16, 16)]
                lo = plsc.bitcast(g << 16, jnp.float32)
                hi = plsc.bitcast(g & jnp.int32(-65536), jnp.float32)
                sl_lo = pl.ds(j * 32, 16)
                sl_hi = pl.ds(j * 32 + 16, 16)
                rows[r, sl_lo] = rows[r, sl_lo] * lo
                rows[r, sl_hi] = rows[r, sl_hi] * hi

        pltpu.async_copy(rows, agg_sh.at[dsts], sems, add=True)

    def _wait_s(dsts, rows, sems):
        pltpu.make_async_copy(rows, agg_sh.at[dsts], sems).wait()

    _issue_c(0, dstb0, srcb0, gate0, semc0)
    _issue_c(1, dstb1, srcb1, gate1, semc1)
    _wait_c(0, dstb0, srcb0, gate0, semc0)
    _issue_g(srcb0, rows0, semg0)

    @pl.loop(0, NCH // 2)
    def _(g):
        a = g * 2
        b = a + 1

        @pl.when(g > 0)
        def _():
            _wait_s(dsts1, rows1, sems1)

        _wait_c(b, dstb1, srcb1, gate1, semc1)
        _issue_g(srcb1, rows1, semg1)

        _wait_g(srcb0, rows0, semg0)
        _proc(dstb0, dsts0, rows0, gate0, sems0)
        _issue_c(a + 2, dstb0, srcb0, gate0, semc0)

        _wait_g(srcb1, rows1, semg1)
        _proc(dstb1, dsts1, rows1, gate1, sems1)

        _wait_s(dsts0, rows0, sems0)
        _wait_c(a + 2, dstb0, srcb0, gate0, semc0)
        _issue_g(srcb0, rows0, semg0)

        @pl.when(b + 2 < NCH)
        def _():
            _issue_c(b + 2, dstb1, srcb1, gate1, semc1)

    _wait_g(srcb0, rows0, semg0)
    _proc(dstb0, dsts0, rows0, gate0, sems0)
    _wait_s(dsts0, rows0, sems0)
    _wait_s(dsts1, rows1, sems1)

    plsc.subcore_barrier()
    pltpu.sync_copy(agg_sh.at[pl.ds(pl.multiple_of(sid * RPT_P, 8), RPT_P)],
                    agg_hbm.at[pl.ds(pl.multiple_of(cid * N_PAD + sid * RPT_P, 8), RPT_P)])


@functools.partial(
    pl.kernel,
    out_type=jax.ShapeDtypeStruct((E, HBS), jnp.float32),
    mesh=_mesh,
    scratch_types=[
        pltpu.VMEM((NCH, CH), jnp.int32),
        pltpu.VMEM((NCH, CH), jnp.int32),
        pltpu.VMEM((CH, HS), jnp.float32),
        pltpu.VMEM((CH, HS), jnp.float32),
        pltpu.VMEM((CH, HS), jnp.float32),
        pltpu.VMEM((CH, HS), jnp.float32),
        pltpu.VMEM((CH, HBS), jnp.float32),
        pltpu.VMEM((CH, HBS), jnp.float32),
        pltpu.SemaphoreType.DMA,
        pltpu.SemaphoreType.DMA,
        pltpu.SemaphoreType.DMA,
    ],
    compiler_params=pltpu.CompilerParams(needs_layout_passes=False),
)
def _pair(hij_hbm, dst_hbm, src_hbm, out_hbm,
          dst2_v, src2_v, ra0, rb0, ra1, rb1, out0, out1, semg0, semg1, semo):
    cid = lax.axis_index("c")
    sid = lax.axis_index("s")
    wid = cid * NS + sid
    base_e = wid * EPW

    pltpu.sync_copy(dst_hbm.at[wid], dst2_v)
    pltpu.sync_copy(src_hbm.at[wid], src2_v)

    def _issue(i, ra, rb, semg):
        pltpu.async_copy(hij_hbm.at[dst2_v.at[i]], ra, semg)
        pltpu.async_copy(hij_hbm.at[src2_v.at[i]], rb, semg)

    def _wait(i, ra, rb, semg):
        pltpu.make_async_copy(hij_hbm.at[dst2_v.at[i]], ra, semg).wait()
        pltpu.make_async_copy(hij_hbm.at[src2_v.at[i]], rb, semg).wait()

    def _proc(i, ra, rb, out, drain):
        off = pl.multiple_of(base_e + i * CH, 8)
        if drain:
            pltpu.make_async_copy(out, out_hbm.at[pl.ds(off, CH)], semo).wait()

        @pl.loop(0, CH)
        def _(r):
            for j in range(HBS // 16):
                sl = pl.ds(j * 16, 16)
                out[r, sl] = ra[r, sl] + rb[r, sl]

        pltpu.async_copy(out, out_hbm.at[pl.ds(off, CH)], semo)

    _issue(0, ra0, rb0, semg0)
    _issue(1, ra1, rb1, semg1)

    _wait(0, ra0, rb0, semg0)
    _proc(0, ra0, rb0, out0, drain=False)
    _issue(2, ra0, rb0, semg0)
    _wait(1, ra1, rb1, semg1)
    _proc(1, ra1, rb1, out1, drain=False)
    _issue(3, ra1, rb1, semg1)

    @pl.loop(1, NCH // 2)
    def _(g):
        i0 = g * 2
        i1 = i0 + 1
        _wait(i0, ra0, rb0, semg0)
        _proc(i0, ra0, rb0, out0, drain=True)

        @pl.when(i0 + 2 < NCH)
        def _():
            _issue(i0 + 2, ra0, rb0, semg0)

        _wait(i1, ra1, rb1, semg1)
        _proc(i1, ra1, rb1, out1, drain=True)

        @pl.when(i1 + 2 < NCH)
        def _():
            _issue(i1 + 2, ra1, rb1, semg1)

    _wait(NCH - 1, ra0, rb0, semg0)
    _proc(NCH - 1, ra0, rb0, out0, drain=True)

    pltpu.make_async_copy(
        out1, out_hbm.at[pl.ds(pl.multiple_of(base_e, 8), CH)], semo).wait()
    pltpu.make_async_copy(
        out0, out_hbm.at[pl.ds(pl.multiple_of(base_e, 8), CH)], semo).wait()


BN = 2000
BE = 4000


def _embed_body(an_ref, table_ref, wmsg_ref, h_ref, hw_ref):
    ids = an_ref[...]
    tt = lax.broadcasted_iota(jnp.int32, (1, NUM_TYPES), 1)
    oh = (ids == tt).astype(jnp.float32)
    h = jnp.dot(oh, table_ref[...], preferred_element_type=jnp.float32)
    h_ref[...] = h
    hw_ref[...] = jnp.dot(h, wmsg_ref[...], preferred_element_type=jnp.float32)


_LOG2E = 1.4426950408889634


def _silu(x):
    return x / (1.0 + jnp.exp2(x * -_LOG2E))


def _rbf_body(d2_ref, lb_ref, kv_ref, rbf_ref, cut_ref):
    d2 = d2_ref[...][:, 0]
    d = jnp.sqrt(d2 + 1e-12)
    x = jnp.exp(-0.5 * d)
    logx = jnp.maximum(-0.5 * d, math.log(1e-10))
    log1mx = jnp.log(jnp.clip(1.0 - x, 1e-10, 1.0))
    lb = lb_ref[...]
    kv = kv_ref[...]
    rbf_ref[...] = jnp.exp(lb + logx[:, None] * kv
                           + log1mx[:, None] * (float(RBF_DIM) - 1.0 - kv))
    t = jnp.clip(d / MAX_RADIUS, 0.0, 1.0)
    sx = jnp.pi * (t - 0.5)
    x2 = sx * sx
    s = sx * (1.0 + x2 * (-1.0 / 6.0 + x2 * (1.0 / 120.0
              + x2 * (-1.0 / 5040.0 + x2 * (1.0 / 362880.0)))))
    cut_ref[...] = (0.5 - 0.5 * s)[:, None]


def _gates_body(rbf_ref, cut_ref, w_ref, gate_ref):
    pre = jnp.dot(rbf_ref[...], w_ref[...], preferred_element_type=jnp.float32)
    gate_ref[...] = (_silu(pre) * cut_ref[...]).astype(jnp.bfloat16)


def _update_body(aggA_ref, aggB_ref, h_ref, wupd_ref, wnext_ref, hn_ref, hw_ref):
    agg = aggA_ref[...] + aggB_ref[...]
    u = jnp.dot(agg, wupd_ref[...], preferred_element_type=jnp.float32)
    hn = h_ref[...] + _silu(u)
    hn_ref[...] = hn
    hw_ref[...] = jnp.dot(hn, wnext_ref[...], preferred_element_type=jnp.float32)


def _final_body(aggA_ref, aggB_ref, h_ref, wupd_ref, wii_ref, wij_ref,
                fii_ref, hij_ref):
    agg = aggA_ref[...] + aggB_ref[...]
    u = jnp.dot(agg, wupd_ref[...], preferred_element_type=jnp.float32)
    hn = h_ref[...] + _silu(u)
    fii_ref[...] = jnp.dot(hn, wii_ref[...], preferred_element_type=jnp.float32)
    hij_ref[...] = jnp.dot(hn, wij_ref[...], preferred_element_type=jnp.float32)


def _node_spec():
    return pl.BlockSpec((BN, HS), lambda i: (i, 0))


def _full(shape):
    return pl.BlockSpec(shape, lambda i: tuple(0 for _ in shape))


_embed = pl.pallas_call(
    _embed_body,
    grid=(N // BN,),
    in_specs=[pl.BlockSpec((BN, 1), lambda i: (i, 0)),
              _full((NUM_TYPES, HS)), _full((HS, HS))],
    out_specs=[_node_spec(), _node_spec()],
    out_shape=[jax.ShapeDtypeStruct((N, HS), jnp.float32),
               jax.ShapeDtypeStruct((N, HS), jnp.float32)],
)

_rbf = pl.pallas_call(
    _rbf_body,
    grid=(E // BE,),
    in_specs=[pl.BlockSpec((BE, 1), lambda i: (i, 0)),
              _full((1, RBF_DIM)), _full((1, RBF_DIM))],
    out_specs=[pl.BlockSpec((BE, RBF_DIM), lambda i: (i, 0)),
               pl.BlockSpec((BE, 1), lambda i: (i, 0))],
    out_shape=[jax.ShapeDtypeStruct((E, RBF_DIM), jnp.float32),
               jax.ShapeDtypeStruct((E, 1), jnp.float32)],
)

_gates = pl.pallas_call(
    _gates_body,
    grid=(E // BE,),
    in_specs=[pl.BlockSpec((BE, RBF_DIM), lambda i: (i, 0)),
              pl.BlockSpec((BE, 1), lambda i: (i, 0)),
              _full((RBF_DIM, HS))],
    out_specs=pl.BlockSpec((BE, HS), lambda i: (i, 0)),
    out_shape=jax.ShapeDtypeStruct((E, HS), jnp.bfloat16),
)

_update = pl.pallas_call(
    _update_body,
    grid=(N // BN,),
    in_specs=[_node_spec(), _node_spec(), _node_spec(),
              _full((HS, HS)), _full((HS, HS))],
    out_specs=[_node_spec(), _node_spec()],
    out_shape=[jax.ShapeDtypeStruct((N, HS), jnp.float32),
               jax.ShapeDtypeStruct((N, HS), jnp.float32)],
)

_final = pl.pallas_call(
    _final_body,
    grid=(N // BN,),
    in_specs=[_node_spec(), _node_spec(), _node_spec(),
              _full((HS, HS)), _full((HS, HBS)), _full((HS, HS))],
    out_specs=[pl.BlockSpec((BN, HBS), lambda i: (i, 0)),
               pl.BlockSpec((BN, HS), lambda i: (i, 0))],
    out_shape=[jax.ShapeDtypeStruct((N, HBS), jnp.float32),
               jax.ShapeDtypeStruct((N, HS), jnp.float32)],
)

_PERM = np.concatenate(
    [np.concatenate([np.arange(32 * j, 32 * (j + 1), 2),
                     np.arange(32 * j + 1, 32 * (j + 1), 2)])
     for j in range(HS // 32)]).astype(np.int32)

_LOGBINOM = np.array(
    [[math.lgamma(RBF_DIM) - math.lgamma(k + 1.0) - math.lgamma(RBF_DIM - k)
      for k in range(RBF_DIM)]], dtype=np.float32)
_KVEC = np.arange(RBF_DIM, dtype=np.float32)[None, :]


def kernel(pos, atomic_numbers, edge_index, table, W_rbf, W_msg, W_upd, W_ii, W_ij):
    W_msg = W_msg[:, :, _PERM]
    W_upd = W_upd[:, _PERM, :]
    dst = edge_index[0].astype(jnp.int32)
    src = edge_index[1].astype(jnp.int32)
    dst3 = dst.reshape(NW, NCH, CH)
    src3 = src.reshape(NW, NCH, CH)
    an2 = atomic_numbers.astype(jnp.int32).reshape(N, 1)

    d2 = _geom(pos.T.reshape(-1), dst, src)
    h, hw = _embed(an2, table, W_msg[0])
    rbf, cut = _rbf(d2.reshape(E, 1), jnp.asarray(_LOGBINOM), jnp.asarray(_KVEC))

    def _gate_i32(g):
        return lax.bitcast_convert_type(
            g.reshape(E, GW, 2), jnp.int32).reshape(E * GW)

    gate = _gates(rbf, cut, W_rbf[0])
    for l in range(NUM_LAYERS):
        aggp = _gms(hw, _gate_i32(gate), dst, src)
        if l < NUM_LAYERS - 1:
            gate = _gates(rbf, cut, W_rbf[l + 1])
        aggA, aggB = aggp[:N], aggp[N_PAD:N_PAD + N]
        if l < NUM_LAYERS - 1:
            h, hw = _update(aggA, aggB, h, W_upd[l], W_msg[l + 1])
        else:
            wij_pad = jnp.pad(W_ij, ((0, 0), (0, HS - HBS)))
            fii, hij = _final(aggA, aggB, h, W_upd[l], W_ii, wij_pad)

    fij = _pair(hij, dst3, src3)
    return jnp.concatenate([fii, fij], axis=0)


# speedup vs baseline: 3.2395x; 3.2395x over previous
import functools
import math

import jax
import jax.numpy as jnp
import numpy as np
from jax import lax
from jax.experimental import pallas as pl
from jax.experimental.pallas import tpu as pltpu
from jax.experimental.pallas import tpu_sc as plsc

HS = 128
HBS = 32
RBF_DIM = 32
MAX_RADIUS = 15.0
NUM_LAYERS = 5
NUM_TYPES = 20
N = 10000
E = 320000

NC = 2
NS = 16
NW = NC * NS
EPW = E // NW
CH = 80
GW = HS // 2
NCH = EPW // CH
RPT = N // NS

_mesh = plsc.VectorSubcoreMesh(core_axis_name="c", subcore_axis_name="s")


@functools.partial(
    pl.kernel,
    out_type=jax.ShapeDtypeStruct((E,), jnp.float32),
    mesh=_mesh,
    scratch_types=[
        pltpu.VMEM((3 * N,), jnp.float32),
        pltpu.VMEM((EPW,), jnp.int32),
        pltpu.VMEM((EPW,), jnp.int32),
        pltpu.VMEM((EPW,), jnp.float32),
    ],
    compiler_params=pltpu.CompilerParams(needs_layout_passes=False),
)
def _geom(pos_hbm, dst_hbm, src_hbm, d2_hbm, pos_v, dst_v, src_v, d2_v):
    cid = lax.axis_index("c")
    sid = lax.axis_index("s")
    base = pl.multiple_of((cid * NS + sid) * EPW, 8)
    pltpu.sync_copy(pos_hbm, pos_v)
    pltpu.sync_copy(dst_hbm.at[pl.ds(base, EPW)], dst_v)
    pltpu.sync_copy(src_hbm.at[pl.ds(base, EPW)], src_v)

    @pl.loop(0, EPW // 16)
    def _(i):
        s = pl.ds(i * 16, 16)
        vd = dst_v[s]
        vs = src_v[s]
        dx = plsc.load_gather(pos_v, [vd]) - plsc.load_gather(pos_v, [vs])
        vd = vd + N
        vs = vs + N
        dy = plsc.load_gather(pos_v, [vd]) - plsc.load_gather(pos_v, [vs])
        vd = vd + N
        vs = vs + N
        dz = plsc.load_gather(pos_v, [vd]) - plsc.load_gather(pos_v, [vs])
        d2_v[s] = dx * dx + dy * dy + dz * dz

    pltpu.sync_copy(d2_v, d2_hbm.at[pl.ds(base, EPW)])


N_PAD = 10240
RPT_P = N_PAD // NS


@functools.partial(
    pl.kernel,
    out_type=jax.ShapeDtypeStruct((NC * N_PAD, HS), jnp.float32),
    mesh=_mesh,
    scratch_types=[
        pltpu.VMEM_SHARED((N_PAD, HS), jnp.float32),
        pltpu.VMEM((CH,), jnp.int32),
        pltpu.VMEM((CH,), jnp.int32),
        pltpu.VMEM((CH,), jnp.int32),
        pltpu.VMEM((CH,), jnp.int32),
        pltpu.VMEM((CH,), jnp.int32),
        pltpu.VMEM((CH,), jnp.int32),
        pltpu.VMEM((CH, HS), jnp.float32),
        pltpu.VMEM((CH, HS), jnp.float32),
        pltpu.VMEM((CH, HS), jnp.float32),
        pltpu.VMEM((CH, HS), jnp.float32),
        pltpu.SemaphoreType.DMA,
        pltpu.SemaphoreType.DMA,
        pltpu.SemaphoreType.DMA,
        pltpu.SemaphoreType.DMA,
        pltpu.SemaphoreType.DMA,
        pltpu.SemaphoreType.DMA,
    ],
    compiler_params=pltpu.CompilerParams(needs_layout_passes=False),
)
def _gms(hw_hbm, gate_hbm, dst_hbm, src_hbm, agg_hbm,
         agg_sh, dstb0, dstb1, srcb0, srcb1, dsts0, dsts1,
         rows0, rows1, gate0, gate1,
         semg0, semg1, semc0, semc1, sems0, sems1):
    cid = lax.axis_index("c")
    sid = lax.axis_index("s")
    wid = cid * NS + sid
    base_e = wid * EPW

    def _issue_c(i, dstb, srcb, gate, semc):
        off = pl.multiple_of(base_e + i * CH, 8)
        pltpu.async_copy(dst_hbm.at[pl.ds(off, CH)], dstb, semc)
        pltpu.async_copy(src_hbm.at[pl.ds(off, CH)], srcb, semc)
        pltpu.async_copy(gate_hbm.at[pl.ds(off, CH)], gate, semc)

    def _wait_c(i, dstb, srcb, gate, semc):
        off = pl.multiple_of(base_e + i * CH, 8)
        pltpu.make_async_copy(dst_hbm.at[pl.ds(off, CH)], dstb, semc).wait()
        pltpu.make_async_copy(src_hbm.at[pl.ds(off, CH)], srcb, semc).wait()
        pltpu.make_async_copy(gate_hbm.at[pl.ds(off, CH)], gate, semc).wait()

    def _issue_g(srcb, rows, semg):
        pltpu.async_copy(hw_hbm.at[srcb], rows, semg)

    def _wait_g(srcb, rows, semg):
        pltpu.make_async_copy(hw_hbm.at[srcb], rows, semg).wait()

    def _proc(dstb, dsts, rows, gate, sems):
        for j in range(CH // 16):
            sl = pl.ds(j * 16, 16)
            dsts[sl] = dstb[sl]

        @pl.loop(0, CH)
        def _(r):
            for j in range(HS // 16):
                sl = pl.ds(j * 16, 16)
                rows[r, sl] = rows[r, sl] * gate[r, sl]

        pltpu.async_copy(rows, agg_sh.at[dsts], sems, add=True)

    def _wait_s(dsts, rows, sems):
        pltpu.make_async_copy(rows, agg_sh.at[dsts], sems).wait()

    _issue_c(0, dstb0, srcb0, gate0, semc0)
    _issue_c(1, dstb1, srcb1, gate1, semc1)

    @pl.loop(0, CH)
    def _(r):
        for j in range(HS // 16):
            rows0[r, pl.ds(j * 16, 16)] = jnp.zeros((16,), jnp.float32)

    for i in range(RPT_P // CH):
        pltpu.sync_copy(
            rows0, agg_sh.at[pl.ds(pl.multiple_of(sid * RPT_P + i * CH, 8), CH)])
    plsc.subcore_barrier()

    _wait_c(0, dstb0, srcb0, gate0, semc0)
    _issue_g(srcb0, rows0, semg0)

    @pl.loop(0, NCH // 2)
    def _(g):
        a = g * 2
        b = a + 1

        @pl.when(g > 0)
        def _():
            _wait_s(dsts1, rows1, sems1)

        _wait_c(b, dstb1, srcb1, gate1, semc1)
        _issue_g(srcb1, rows1, semg1)

        _wait_g(srcb0, rows0, semg0)
        _proc(dstb0, dsts0, rows0, gate0, sems0)
        _issue_c(a + 2, dstb0, srcb0, gate0, semc0)

        _wait_g(srcb1, rows1, semg1)
        _proc(dstb1, dsts1, rows1, gate1, sems1)

        _wait_s(dsts0, rows0, sems0)
        _wait_c(a + 2, dstb0, srcb0, gate0, semc0)
        _issue_g(srcb0, rows0, semg0)

        @pl.when(b + 2 < NCH)
        def _():
            _issue_c(b + 2, dstb1, srcb1, gate1, semc1)

    _wait_g(srcb0, rows0, semg0)
    _proc(dstb0, dsts0, rows0, gate0, sems0)
    _wait_s(dsts0, rows0, sems0)
    _wait_s(dsts1, rows1, sems1)

    plsc.subcore_barrier()
    pltpu.sync_copy(agg_sh.at[pl.ds(pl.multiple_of(sid * RPT_P, 8), RPT_P)],
                    agg_hbm.at[pl.ds(pl.multiple_of(cid * N_PAD + sid * RPT_P, 8), RPT_P)])


@functools.partial(
    pl.kernel,
    out_type=jax.ShapeDtypeStruct((E, HBS), jnp.float32),
    mesh=_mesh,
    scratch_types=[
        pltpu.VMEM((NCH, CH), jnp.int32),
        pltpu.VMEM((NCH, CH), jnp.int32),
        pltpu.VMEM((CH, HS), jnp.float32),
        pltpu.VMEM((CH, HS), jnp.float32),
        pltpu.VMEM((CH, HS), jnp.float32),
        pltpu.VMEM((CH, HS), jnp.float32),
        pltpu.VMEM((CH, HBS), jnp.float32),
        pltpu.VMEM((CH, HBS), jnp.float32),
        pltpu.SemaphoreType.DMA,
        pltpu.SemaphoreType.DMA,
        pltpu.SemaphoreType.DMA,
    ],
    compiler_params=pltpu.CompilerParams(needs_layout_passes=False),
)
def _pair(hij_hbm, dst_hbm, src_hbm, out_hbm,
          dst2_v, src2_v, ra0, rb0, ra1, rb1, out0, out1, semg0, semg1, semo):
    cid = lax.axis_index("c")
    sid = lax.axis_index("s")
    wid = cid * NS + sid
    base_e = wid * EPW

    pltpu.sync_copy(dst_hbm.at[wid], dst2_v)
    pltpu.sync_copy(src_hbm.at[wid], src2_v)

    def _issue(i, ra, rb, semg):
        pltpu.async_copy(hij_hbm.at[dst2_v.at[i]], ra, semg)
        pltpu.async_copy(hij_hbm.at[src2_v.at[i]], rb, semg)

    def _wait(i, ra, rb, semg):
        pltpu.make_async_copy(hij_hbm.at[dst2_v.at[i]], ra, semg).wait()
        pltpu.make_async_copy(hij_hbm.at[src2_v.at[i]], rb, semg).wait()

    def _proc(i, ra, rb, out, drain):
        off = pl.multiple_of(base_e + i * CH, 8)
        if drain:
            pltpu.make_async_copy(out, out_hbm.at[pl.ds(off, CH)], semo).wait()

        @pl.loop(0, CH)
        def _(r):
            for j in range(HBS // 16):
                sl = pl.ds(j * 16, 16)
                out[r, sl] = ra[r, sl] + rb[r, sl]

        pltpu.async_copy(out, out_hbm.at[pl.ds(off, CH)], semo)

    _issue(0, ra0, rb0, semg0)
    _issue(1, ra1, rb1, semg1)

    _wait(0, ra0, rb0, semg0)
    _proc(0, ra0, rb0, out0, drain=False)
    _issue(2, ra0, rb0, semg0)
    _wait(1, ra1, rb1, semg1)
    _proc(1, ra1, rb1, out1, drain=False)
    _issue(3, ra1, rb1, semg1)

    @pl.loop(1, NCH // 2)
    def _(g):
        i0 = g * 2
        i1 = i0 + 1
        _wait(i0, ra0, rb0, semg0)
        _proc(i0, ra0, rb0, out0, drain=True)

        @pl.when(i0 + 2 < NCH)
        def _():
            _issue(i0 + 2, ra0, rb0, semg0)

        _wait(i1, ra1, rb1, semg1)
        _proc(i1, ra1, rb1, out1, drain=True)

        @pl.when(i1 + 2 < NCH)
        def _():
            _issue(i1 + 2, ra1, rb1, semg1)

    _wait(NCH - 1, ra0, rb0, semg0)
    _proc(NCH - 1, ra0, rb0, out0, drain=True)

    pltpu.make_async_copy(
        out1, out_hbm.at[pl.ds(pl.multiple_of(base_e, 8), CH)], semo).wait()
    pltpu.make_async_copy(
        out0, out_hbm.at[pl.ds(pl.multiple_of(base_e, 8), CH)], semo).wait()


BN = 2000
BE = 4000


def _embed_body(an_ref, table_ref, wmsg_ref, h_ref, hw_ref):
    ids = an_ref[...]
    tt = lax.broadcasted_iota(jnp.int32, (1, NUM_TYPES), 1)
    oh = (ids == tt).astype(jnp.float32)
    h = jnp.dot(oh, table_ref[...], preferred_element_type=jnp.float32)
    h_ref[...] = h
    hw_ref[...] = jnp.dot(h, wmsg_ref[...], preferred_element_type=jnp.float32)


_LOG2E = 1.4426950408889634


def _silu(x):
    return x / (1.0 + jnp.exp2(x * -_LOG2E))


def _gates_body(d2_ref, lb_ref, kv_ref, w_ref, gate_ref):
    d2 = d2_ref[...][:, 0]
    d = jnp.sqrt(d2 + 1e-12)
    x = jnp.exp(-0.5 * d)
    logx = jnp.maximum(-0.5 * d, math.log(1e-10))
    log1mx = jnp.log(jnp.clip(1.0 - x, 1e-10, 1.0))
    lb = lb_ref[...]
    kv = kv_ref[...]
    rbf = jnp.exp(lb + logx[:, None] * kv
                  + log1mx[:, None] * (float(RBF_DIM) - 1.0 - kv))
    t = jnp.clip(d / MAX_RADIUS, 0.0, 1.0)
    sx = jnp.pi * (t - 0.5)
    x2 = sx * sx
    s = sx * (1.0 + x2 * (-1.0 / 6.0 + x2 * (1.0 / 120.0
              + x2 * (-1.0 / 5040.0 + x2 * (1.0 / 362880.0)))))
    cut = 0.5 - 0.5 * s
    pre = jnp.dot(rbf, w_ref[...], preferred_element_type=jnp.float32)
    gate_ref[...] = _silu(pre) * cut[:, None]


def _update_body(aggA_ref, aggB_ref, h_ref, wupd_ref, wnext_ref, hn_ref, hw_ref):
    agg = aggA_ref[...] + aggB_ref[...]
    u = jnp.dot(agg, wupd_ref[...], preferred_element_type=jnp.float32)
    hn = h_ref[...] + _silu(u)
    hn_ref[...] = hn
    hw_ref[...] = jnp.dot(hn, wnext_ref[...], preferred_element_type=jnp.float32)


def _final_body(aggA_ref, aggB_ref, h_ref, wupd_ref, wii_ref, wij_ref,
                fii_ref, hij_ref):
    agg = aggA_ref[...] + aggB_ref[...]
    u = jnp.dot(agg, wupd_ref[...], preferred_element_type=jnp.float32)
    hn = h_ref[...] + _silu(u)
    fii_ref[...] = jnp.dot(hn, wii_ref[...], preferred_element_type=jnp.float32)
    hij_ref[...] = jnp.dot(hn, wij_ref[...], preferred_element_type=jnp.float32)


def _node_spec():
    return pl.BlockSpec((BN, HS), lambda i: (i, 0))


def _full(shape):
    return pl.BlockSpec(shape, lambda i: tuple(0 for _ in shape))


_embed = pl.pallas_call(
    _embed_body,
    grid=(N // BN,),
    in_specs=[pl.BlockSpec((BN, 1), lambda i: (i, 0)),
              _full((NUM_TYPES, HS)), _full((HS, HS))],
    out_specs=[_node_spec(), _node_spec()],
    out_shape=[jax.ShapeDtypeStruct((N, HS), jnp.float32),
               jax.ShapeDtypeStruct((N, HS), jnp.float32)],
)

_gates = pl.pallas_call(
    _gates_body,
    grid=(E // BE,),
    in_specs=[pl.BlockSpec((BE, 1), lambda i: (i, 0)),
              _full((1, RBF_DIM)), _full((1, RBF_DIM)),
              _full((RBF_DIM, HS))],
    out_specs=pl.BlockSpec((BE, HS), lambda i: (i, 0)),
    out_shape=jax.ShapeDtypeStruct((E, HS), jnp.float32),
)

_update = pl.pallas_call(
    _update_body,
    grid=(N // BN,),
    in_specs=[_node_spec(), _node_spec(), _node_spec(),
              _full((HS, HS)), _full((HS, HS))],
    out_specs=[_node_spec(), _node_spec()],
    out_shape=[jax.ShapeDtypeStruct((N, HS), jnp.float32),
               jax.ShapeDtypeStruct((N, HS), jnp.float32)],
)

_final = pl.pallas_call(
    _final_body,
    grid=(N // BN,),
    in_specs=[_node_spec(), _node_spec(), _node_spec(),
              _full((HS, HS)), _full((HS, HBS)), _full((HS, HS))],
    out_specs=[pl.BlockSpec((BN, HBS), lambda i: (i, 0)),
               pl.BlockSpec((BN, HS), lambda i: (i, 0))],
    out_shape=[jax.ShapeDtypeStruct((N, HBS), jnp.float32),
               jax.ShapeDtypeStruct((N, HS), jnp.float32)],
)

_LOGBINOM = np.array(
    [[math.lgamma(RBF_DIM) - math.lgamma(k + 1.0) - math.lgamma(RBF_DIM - k)
      for k in range(RBF_DIM)]], dtype=np.float32)
_KVEC = np.arange(RBF_DIM, dtype=np.float32)[None, :]


def kernel(pos, atomic_numbers, edge_index, table, W_rbf, W_msg, W_upd, W_ii, W_ij):
    dst = edge_index[0].astype(jnp.int32)
    src = edge_index[1].astype(jnp.int32)
    dst3 = dst.reshape(NW, NCH, CH)
    src3 = src.reshape(NW, NCH, CH)
    an2 = atomic_numbers.astype(jnp.int32).reshape(N, 1)

    d2 = _geom(pos.T.reshape(-1), dst, src)
    h, hw = _embed(an2, table, W_msg[0])
    d2r = d2.reshape(E, 1)
    lb = jnp.asarray(_LOGBINOM)
    kv = jnp.asarray(_KVEC)

    gate = _gates(d2r, lb, kv, W_rbf[0])
    for l in range(NUM_LAYERS):
        aggp = _gms(hw, gate, dst, src)
        if l < NUM_LAYERS - 1:
            gate = _gates(d2r, lb, kv, W_rbf[l + 1])
        aggA, aggB = aggp[:N], aggp[N_PAD:N_PAD + N]
        if l < NUM_LAYERS - 1:
            h, hw = _update(aggA, aggB, h, W_upd[l], W_msg[l + 1])
        else:
            wij_pad = jnp.pad(W_ij, ((0, 0), (0, HS - HBS)))
            fii, hij = _final(aggA, aggB, h, W_upd[l], W_ii, wij_pad)

    fij = _pair(hij, dst3, src3)
    return jnp.concatenate([fii, fij], axis=0)
